# baseline (speedup 1.0000x reference)
import functools

import jax
import jax.numpy as jnp
from jax import lax
from jax.experimental import pallas as pl
from jax.experimental.pallas import tpu as pltpu
from jax.experimental.pallas import tpu_sc as plsc

DIM = 32
PDIM = 128
OUT_W = 16
L = 50

NUM_CORES = 2
NUM_SUBCORES = 16
NW = NUM_CORES * NUM_SUBCORES

BAGS_PER_W = 512
TOK_PER_W = BAGS_PER_W * L
SUP_BAGS = 8
SUP_TOK = SUP_BAGS * L
N_SUP = BAGS_PER_W // SUP_BAGS
DMA_SPLITS = [(0, 128), (128, 128), (256, 128), (384, 16)]

PROJ_BLK = 2048


def _project_tc(table_t, wp):
    vocab = table_t.shape[1]
    grid = (vocab + PROJ_BLK - 1) // PROJ_BLK

    def body(t_ref, w_ref, o_ref):
        o_ref[...] = lax.dot_general(
            t_ref[...], w_ref[...], (((0,), (0,)), ((), ())),
            preferred_element_type=jnp.float32,
            precision=lax.Precision.HIGHEST,
        )

    return pl.pallas_call(
        body,
        grid=(grid,),
        in_specs=[
            pl.BlockSpec((DIM, PROJ_BLK), lambda i: (0, i)),
            pl.BlockSpec((DIM, PDIM), lambda i: (0, 0)),
        ],
        out_specs=pl.BlockSpec((PROJ_BLK, PDIM), lambda i: (i, 0)),
        out_shape=jax.ShapeDtypeStruct((grid * PROJ_BLK, PDIM), jnp.float32),
    )(table_t, wp)


def _bag_sums_sc(text, proj, b16, n_bags):
    mesh = plsc.VectorSubcoreMesh(core_axis_name="c", subcore_axis_name="s")

    @functools.partial(
        pl.kernel,
        out_type=jax.ShapeDtypeStruct((n_bags, OUT_W), jnp.float32),
        mesh=mesh,
        compiler_params=pltpu.CompilerParams(use_tc_tiling_on_sc=False),
        scratch_types=[
            pltpu.VMEM((TOK_PER_W,), jnp.int32),
            pltpu.VMEM((SUP_TOK, PDIM), jnp.float32),
            pltpu.VMEM((SUP_BAGS, OUT_W), jnp.float32),
            pltpu.VMEM((OUT_W,), jnp.float32),
            pltpu.SemaphoreType.DMA,
            pltpu.SemaphoreType.DMA,
        ],
    )
    def k(text_hbm, proj_hbm, b16_hbm, out_hbm, idx_v, rows_v, sums_v, b_v,
          gsem, osem):
        wid = lax.axis_index("s") * NUM_CORES + lax.axis_index("c")
        tok_base = wid * TOK_PER_W

        pltpu.sync_copy(b16_hbm, b_v)
        pltpu.sync_copy(text_hbm.at[pl.ds(tok_base, TOK_PER_W)], idx_v)

        bias = b_v[...]

        def superchunk(s, _):
            s_tok = pl.multiple_of(s * SUP_TOK, 8)
            copies = []
            for off, n in DMA_SPLITS:
                o = pl.multiple_of(s_tok + off, 8)
                copies.append(pltpu.async_copy(
                    proj_hbm.at[idx_v.at[pl.ds(o, n)]],
                    rows_v.at[pl.ds(off, n)],
                    gsem,
                ))
            for c in copies:
                c.wait()

            def bag_body(bag, _):
                tok0 = bag * L
                acc = bias
                for t in range(L):
                    acc = acc + rows_v[tok0 + t, 0:OUT_W]
                sums_v[bag, :] = acc
                return _

            lax.fori_loop(0, SUP_BAGS, bag_body, None)

            row0 = wid * BAGS_PER_W + s * SUP_BAGS
            pltpu.async_copy(
                sums_v, out_hbm.at[pl.ds(row0, SUP_BAGS)], osem
            ).wait()
            return _

        lax.fori_loop(0, N_SUP, superchunk, None)

    return k(text, proj, b16)


def kernel(text, offsets, table, W, b):
    n_bags = offsets.shape[0]
    nclass = W.shape[0]
    wp = jnp.zeros((DIM, PDIM), jnp.float32).at[:, :nclass].set(W.T * (1.0 / L))
    b16 = jnp.zeros((OUT_W,), jnp.float32).at[:nclass].set(b)
    proj = _project_tc(table.T, wp)
    sums = _bag_sums_sc(text, proj, b16, n_bags)
    return sums[:, :nclass]


# speedup vs baseline: 1.6538x; 1.6538x over previous
import functools

import jax
import jax.numpy as jnp
from jax import lax
from jax.experimental import pallas as pl
from jax.experimental.pallas import tpu as pltpu
from jax.experimental.pallas import tpu_sc as plsc

DIM = 32
PDIM = 128
OUT_W = 16
L = 50

NUM_CORES = 2
NUM_SUBCORES = 16
NW = NUM_CORES * NUM_SUBCORES

BAGS_PER_W = 512
TOK_PER_W = BAGS_PER_W * L
SUP_BAGS = 8
SUP_TOK = SUP_BAGS * L
N_SUP = BAGS_PER_W // SUP_BAGS
DMA_SPLITS = [(0, 128), (128, 128), (256, 128), (384, 16)]

PROJ_BLK = 8192


def _project_tc(table_t, wp):
    vocab = table_t.shape[1]
    grid = (vocab + PROJ_BLK - 1) // PROJ_BLK

    def body(t_ref, w_ref, o_ref):
        o_ref[...] = lax.dot_general(
            t_ref[...], w_ref[...], (((0,), (0,)), ((), ())),
            preferred_element_type=jnp.float32,
        )

    return pl.pallas_call(
        body,
        grid=(grid,),
        in_specs=[
            pl.BlockSpec((DIM, PROJ_BLK), lambda i: (0, i)),
            pl.BlockSpec((DIM, PDIM), lambda i: (0, 0)),
        ],
        out_specs=pl.BlockSpec((PROJ_BLK, PDIM), lambda i: (i, 0)),
        out_shape=jax.ShapeDtypeStruct((grid * PROJ_BLK, PDIM), jnp.float32),
    )(table_t, wp)


def _bag_sums_sc(text, proj, b16, n_bags):
    mesh = plsc.VectorSubcoreMesh(core_axis_name="c", subcore_axis_name="s")

    @functools.partial(
        pl.kernel,
        out_type=jax.ShapeDtypeStruct((n_bags, OUT_W), jnp.float32),
        mesh=mesh,
        compiler_params=pltpu.CompilerParams(use_tc_tiling_on_sc=False),
        scratch_types=[
            pltpu.VMEM((TOK_PER_W,), jnp.int32),
            pltpu.VMEM((SUP_TOK, PDIM), jnp.float32),
            pltpu.VMEM((SUP_BAGS, OUT_W), jnp.float32),
            pltpu.VMEM((OUT_W,), jnp.float32),
            pltpu.SemaphoreType.DMA,
            pltpu.SemaphoreType.DMA,
        ],
    )
    def k(text_hbm, proj_hbm, b16_hbm, out_hbm, idx_v, rows_v, sums_v, b_v,
          gsem, osem):
        wid = lax.axis_index("s") * NUM_CORES + lax.axis_index("c")
        tok_base = wid * TOK_PER_W

        pltpu.sync_copy(b16_hbm, b_v)
        pltpu.sync_copy(text_hbm.at[pl.ds(tok_base, TOK_PER_W)], idx_v)

        bias = b_v[...]

        def superchunk(s, _):
            s_tok = pl.multiple_of(s * SUP_TOK, 8)
            copies = []
            for off, n in DMA_SPLITS:
                o = pl.multiple_of(s_tok + off, 8)
                copies.append(pltpu.async_copy(
                    proj_hbm.at[idx_v.at[pl.ds(o, n)]],
                    rows_v.at[pl.ds(off, n)],
                    gsem,
                ))
            for c in copies:
                c.wait()

            def bag_body(bag, _):
                tok0 = bag * L
                acc = bias
                for t in range(L):
                    acc = acc + rows_v[tok0 + t, 0:OUT_W]
                sums_v[bag, :] = acc
                return _

            lax.fori_loop(0, SUP_BAGS, bag_body, None)

            row0 = wid * BAGS_PER_W + s * SUP_BAGS
            pltpu.async_copy(
                sums_v, out_hbm.at[pl.ds(row0, SUP_BAGS)], osem
            ).wait()
            return _

        lax.fori_loop(0, N_SUP, superchunk, None)

    return k(text, proj, b16)


def kernel(text, offsets, table, W, b):
    n_bags = offsets.shape[0]
    nclass = W.shape[0]
    wp = jnp.zeros((DIM, PDIM), jnp.float32).at[:, :nclass].set(W.T * (1.0 / L))
    b16 = jnp.zeros((OUT_W,), jnp.float32).at[:nclass].set(b)
    proj = _project_tc(table.T, wp)
    sums = _bag_sums_sc(text, proj, b16, n_bags)
    return sums[:, :nclass]


# speedup vs baseline: 1.7883x; 1.0814x over previous
import functools

import jax
import jax.numpy as jnp
from jax import lax
from jax.experimental import pallas as pl
from jax.experimental.pallas import tpu as pltpu
from jax.experimental.pallas import tpu_sc as plsc

DIM = 32
PDIM = 128
OUT_W = 16
L = 50

NUM_CORES = 2
NUM_SUBCORES = 16
NW = NUM_CORES * NUM_SUBCORES

BAGS_PER_W = 512
TOK_PER_W = BAGS_PER_W * L
SUP_BAGS = 8
SUP_TOK = SUP_BAGS * L
N_SUP = BAGS_PER_W // SUP_BAGS
DMA_SPLITS = [(0, 128), (128, 128), (256, 128), (384, 16)]

PROJ_BLK = 8192


def _project_tc(table_t, wp):
    vocab = table_t.shape[1]
    grid = (vocab + PROJ_BLK - 1) // PROJ_BLK

    def body(t_ref, w_ref, o_ref):
        o_ref[...] = lax.dot_general(
            t_ref[...], w_ref[...], (((0,), (0,)), ((), ())),
            preferred_element_type=jnp.float32,
        )

    return pl.pallas_call(
        body,
        grid=(grid,),
        in_specs=[
            pl.BlockSpec((DIM, PROJ_BLK), lambda i: (0, i)),
            pl.BlockSpec((DIM, PDIM), lambda i: (0, 0)),
        ],
        out_specs=pl.BlockSpec((PROJ_BLK, PDIM), lambda i: (i, 0)),
        out_shape=jax.ShapeDtypeStruct((grid * PROJ_BLK, PDIM), jnp.float32),
    )(table_t, wp)


def _bag_sums_sc(text, proj, b16, n_bags):
    mesh = plsc.VectorSubcoreMesh(core_axis_name="c", subcore_axis_name="s")

    @functools.partial(
        pl.kernel,
        out_type=jax.ShapeDtypeStruct((n_bags, OUT_W), jnp.float32),
        mesh=mesh,
        compiler_params=pltpu.CompilerParams(use_tc_tiling_on_sc=False),
        scratch_types=[
            pltpu.VMEM((TOK_PER_W,), jnp.int32),
            pltpu.VMEM((2, SUP_TOK, PDIM), jnp.float32),
            pltpu.VMEM((2, SUP_BAGS, OUT_W), jnp.float32),
            pltpu.VMEM((OUT_W,), jnp.float32),
            pltpu.SemaphoreType.DMA,
            pltpu.SemaphoreType.DMA,
            pltpu.SemaphoreType.DMA,
            pltpu.SemaphoreType.DMA,
        ],
    )
    def k(text_hbm, proj_hbm, b16_hbm, out_hbm, idx_v, rows_v, sums_v, b_v,
          gsem0, gsem1, osem0, osem1):
        wid = lax.axis_index("s") * NUM_CORES + lax.axis_index("c")
        tok_base = wid * TOK_PER_W

        pltpu.sync_copy(b16_hbm, b_v)
        pltpu.sync_copy(text_hbm.at[pl.ds(tok_base, TOK_PER_W)], idx_v)

        bias = b_v[...]
        gsems = (gsem0, gsem1)
        osems = (osem0, osem1)

        def fire(s, buf):
            for off, n in DMA_SPLITS:
                o = pl.multiple_of(s * SUP_TOK + off, 8)
                pltpu.async_copy(
                    proj_hbm.at[idx_v.at[pl.ds(o, n)]],
                    rows_v.at[buf, pl.ds(off, n)],
                    gsems[buf],
                )

        def consume(s, buf):
            def bag_body(bag, _):
                tok0 = bag * L
                acc = bias
                for t in range(L):
                    acc = acc + rows_v[buf, tok0 + t, 0:OUT_W]
                sums_v[buf, bag, :] = acc
                return _

            lax.fori_loop(0, SUP_BAGS, bag_body, None)

            row0 = wid * BAGS_PER_W + s * SUP_BAGS
            return pltpu.async_copy(
                sums_v.at[buf], out_hbm.at[pl.ds(row0, SUP_BAGS)], osems[buf]
            )

        def drain(buf):
            for off, n in DMA_SPLITS:
                pltpu.make_async_copy(
                    proj_hbm.at[idx_v.at[pl.ds(off, n)]],
                    rows_v.at[buf, pl.ds(off, n)],
                    gsems[buf],
                ).wait()

        fire(0, 0)

        def pair_body(g, _):
            s0 = g * 2
            fire(s0 + 1, 1)
            drain(0)
            consume(s0, 0).wait()
            fire(s0 + 2, 0)
            drain(1)
            consume(s0 + 1, 1).wait()
            return _

        lax.fori_loop(0, N_SUP // 2 - 1, pair_body, None)
        s0 = N_SUP - 2
        fire(s0 + 1, 1)
        drain(0)
        consume(s0, 0).wait()
        drain(1)
        consume(s0 + 1, 1).wait()

    return k(text, proj, b16)


def kernel(text, offsets, table, W, b):
    n_bags = offsets.shape[0]
    nclass = W.shape[0]
    wp = jnp.zeros((DIM, PDIM), jnp.float32).at[:, :nclass].set(W.T * (1.0 / L))
    b16 = jnp.zeros((OUT_W,), jnp.float32).at[:nclass].set(b)
    proj = _project_tc(table.T, wp)
    sums = _bag_sums_sc(text, proj, b16, n_bags)
    return sums[:, :nclass]


# speedup vs baseline: 1.9278x; 1.0780x over previous
import functools

import jax
import jax.numpy as jnp
from jax import lax
from jax.experimental import pallas as pl
from jax.experimental.pallas import tpu as pltpu
from jax.experimental.pallas import tpu_sc as plsc

DIM = 32
PDIM = 128
OUT_W = 16
L = 50

NUM_CORES = 2
NUM_SUBCORES = 16
NW = NUM_CORES * NUM_SUBCORES

BAGS_PER_W = 512
TOK_PER_W = BAGS_PER_W * L
SUP_BAGS = 8
SUP_TOK = SUP_BAGS * L
N_SUP = BAGS_PER_W // SUP_BAGS
DMA_SPLITS = [(0, 128), (128, 128), (256, 128), (384, 16)]

PROJ_BLK = 16384


def _project_tc(table_t, wp):
    vocab = table_t.shape[1]
    grid = (vocab + PROJ_BLK - 1) // PROJ_BLK

    def body(t_ref, w_ref, o_ref):
        o_ref[...] = lax.dot_general(
            t_ref[...], w_ref[...], (((0,), (0,)), ((), ())),
            preferred_element_type=jnp.float32,
        )

    return pl.pallas_call(
        body,
        grid=(grid,),
        in_specs=[
            pl.BlockSpec((DIM, PROJ_BLK), lambda i: (0, i)),
            pl.BlockSpec((DIM, PDIM), lambda i: (0, 0)),
        ],
        out_specs=pl.BlockSpec((PROJ_BLK, PDIM), lambda i: (i, 0)),
        out_shape=jax.ShapeDtypeStruct((grid * PROJ_BLK, PDIM), jnp.float32),
    )(table_t, wp)


def _bag_sums_sc(text, proj, b16, n_bags):
    mesh = plsc.VectorSubcoreMesh(core_axis_name="c", subcore_axis_name="s")

    @functools.partial(
        pl.kernel,
        out_type=jax.ShapeDtypeStruct((n_bags, OUT_W), jnp.float32),
        mesh=mesh,
        compiler_params=pltpu.CompilerParams(use_tc_tiling_on_sc=False),
        scratch_types=[
            pltpu.VMEM((TOK_PER_W,), jnp.int32),
            pltpu.VMEM((2, SUP_TOK, PDIM), jnp.float32),
            pltpu.VMEM((2, SUP_BAGS, OUT_W), jnp.float32),
            pltpu.VMEM((OUT_W,), jnp.float32),
            pltpu.SemaphoreType.DMA,
            pltpu.SemaphoreType.DMA,
            pltpu.SemaphoreType.DMA,
            pltpu.SemaphoreType.DMA,
        ],
    )
    def k(text_hbm, proj_hbm, b16_hbm, out_hbm, idx_v, rows_v, sums_v, b_v,
          gsem0, gsem1, osem0, osem1):
        wid = lax.axis_index("s") * NUM_CORES + lax.axis_index("c")
        tok_base = wid * TOK_PER_W

        pltpu.sync_copy(b16_hbm, b_v)
        pltpu.sync_copy(text_hbm.at[pl.ds(tok_base, TOK_PER_W)], idx_v)

        bias = b_v[...]
        gsems = (gsem0, gsem1)
        osems = (osem0, osem1)

        def fire(s, buf):
            for off, n in DMA_SPLITS:
                o = pl.multiple_of(s * SUP_TOK + off, 8)
                pltpu.async_copy(
                    proj_hbm.at[idx_v.at[pl.ds(o, n)]],
                    rows_v.at[buf, pl.ds(off, n)],
                    gsems[buf],
                )

        def consume(s, buf):
            def bag_body(bag, _):
                tok0 = bag * L
                acc = bias
                for t in range(L):
                    acc = acc + rows_v[buf, tok0 + t, 0:OUT_W]
                sums_v[buf, bag, :] = acc
                return _

            lax.fori_loop(0, SUP_BAGS, bag_body, None)

            row0 = wid * BAGS_PER_W + s * SUP_BAGS
            return pltpu.async_copy(
                sums_v.at[buf], out_hbm.at[pl.ds(row0, SUP_BAGS)], osems[buf]
            )

        def drain(buf):
            for off, n in DMA_SPLITS:
                pltpu.make_async_copy(
                    proj_hbm.at[idx_v.at[pl.ds(off, n)]],
                    rows_v.at[buf, pl.ds(off, n)],
                    gsems[buf],
                ).wait()

        fire(0, 0)

        def pair_body(g, _):
            s0 = g * 2
            fire(s0 + 1, 1)
            drain(0)
            consume(s0, 0).wait()
            fire(s0 + 2, 0)
            drain(1)
            consume(s0 + 1, 1).wait()
            return _

        lax.fori_loop(0, N_SUP // 2 - 1, pair_body, None)
        s0 = N_SUP - 2
        fire(s0 + 1, 1)
        drain(0)
        consume(s0, 0).wait()
        drain(1)
        consume(s0 + 1, 1).wait()

    return k(text, proj, b16)


def kernel(text, offsets, table, W, b):
    n_bags = offsets.shape[0]
    nclass = W.shape[0]
    wp = jnp.zeros((DIM, PDIM), jnp.float32).at[:, :nclass].set(W.T * (1.0 / L))
    b16 = jnp.zeros((OUT_W,), jnp.float32).at[:nclass].set(b)
    proj = _project_tc(table.T, wp)
    sums = _bag_sums_sc(text, proj, b16, n_bags)
    return sums[:, :nclass]
